# Initial kernel scaffold; baseline (speedup 1.0000x reference)
import jax
import jax.numpy as jnp
from jax.experimental import pallas as pl


def kernel(x, W1, b1, W2, b2, memory_keys, memory_values, Wh1, bh1, Wh2, bh2):
    raise NotImplementedError("write your pallas kernel here")


# speedup vs baseline: 4.3111x; 4.3111x over previous
import functools

import jax
import jax.numpy as jnp
from jax.experimental import pallas as pl

TAU = 0.7

BN = 512


_INV_SQRT2 = 0.7071067811865476


def _gelu(v):
    return 0.5 * v * (1.0 + jax.lax.erf(v * _INV_SQRT2))


def _fused_kernel(x_ref, w1_ref, b1_ref, w2_ref, b2_ref, mk_ref, mv_ref,
                  wh1a_ref, wh1b_ref, bh1_ref, wh2_ref, bh2_ref, out_ref):
    x = x_ref[...]
    z1 = _gelu(jnp.dot(x, w1_ref[...]) + b1_ref[...])
    z = _gelu(jnp.dot(z1, w2_ref[...]) + b2_ref[...])

    logits = jnp.dot(z, mk_ref[...]) * (1.0 / TAU)
    k = logits.shape[-1]
    col = jax.lax.broadcasted_iota(jnp.int32, logits.shape, 1)

    v0 = jnp.max(logits, axis=-1, keepdims=True)
    i0 = jnp.min(jnp.where(logits == v0, col, k), axis=-1, keepdims=True)
    oh0 = (col == i0).astype(jnp.float32)
    masked = jnp.where(col == i0, -jnp.inf, logits)
    v1 = jnp.max(masked, axis=-1, keepdims=True)
    i1 = jnp.min(jnp.where(masked == v1, col, k), axis=-1, keepdims=True)
    oh1 = (col == i1).astype(jnp.float32)

    e = jnp.exp(v1 - v0)
    denom = 1.0 + e
    p0 = 1.0 / denom
    p1 = e / denom
    attn = p0 * oh0 + p1 * oh1
    mem = jnp.dot(attn, mv_ref[...])

    h = _gelu(jnp.dot(z, wh1a_ref[...]) + jnp.dot(mem, wh1b_ref[...])
              + bh1_ref[...])
    head = jnp.dot(h, wh2_ref[...]) + bh2_ref[...]
    out_ref[...] = jax.nn.sigmoid(head)


@jax.jit
def _run(x, W1, b1, W2, b2, memory_keys, memory_values, Wh1, bh1, Wh2, bh2):
    n, d = x.shape
    h0 = W1.shape[1]
    l = W2.shape[1]

    mkT = memory_keys.T
    wh1a = Wh1[:l]
    wh1b = Wh1[l:]

    rep = lambda *shape: pl.BlockSpec(shape, lambda i: (0,) * len(shape))
    out = pl.pallas_call(
        _fused_kernel,
        grid=(n // BN,),
        in_specs=[
            pl.BlockSpec((BN, d), lambda i: (i, 0)),
            rep(d, h0), rep(1, h0), rep(h0, l), rep(1, l),
            rep(l, memory_keys.shape[0]), rep(memory_values.shape[0], l),
            rep(l, h0), rep(l, h0), rep(1, h0), rep(h0, 1), rep(1, 1),
        ],
        out_specs=pl.BlockSpec((BN, 1), lambda i: (i, 0)),
        out_shape=jax.ShapeDtypeStruct((n, 1), jnp.float32),
    )(x, W1, b1.reshape(1, -1), W2, b2.reshape(1, -1), mkT, memory_values,
      wh1a, wh1b, bh1.reshape(1, -1), Wh2, bh2.reshape(1, -1))
    return out.reshape(n)


def kernel(x, W1, b1, W2, b2, memory_keys, memory_values, Wh1, bh1, Wh2, bh2):
    return _run(x, W1, b1, W2, b2, memory_keys, memory_values,
                Wh1, bh1, Wh2, bh2)


# speedup vs baseline: 4.7072x; 1.0919x over previous
import functools

import jax
import jax.numpy as jnp
from jax.experimental import pallas as pl

TAU = 0.7

BN = 512


_INV_SQRT2 = 0.7071067811865476


def _gelu(v):
    return 0.5 * v * (1.0 + jax.lax.erf(v * _INV_SQRT2))


def _fused_kernel(x_ref, w1_ref, b1_ref, w2_ref, b2_ref, mk_ref, mv_ref,
                  wh1a_ref, wh1b_ref, bh1_ref, wh2_ref, bh2_ref, out_ref):
    x = x_ref[...]
    z1 = _gelu(jnp.dot(x, w1_ref[...]) + b1_ref[...])
    z = _gelu(jnp.dot(z1, w2_ref[...]) + b2_ref[...])

    logits = jnp.dot(z, mk_ref[...]) * (1.0 / TAU)

    v0 = jnp.max(logits, axis=-1, keepdims=True)
    eq0 = (logits == v0).astype(jnp.float32)
    masked = logits - eq0 * jnp.float32(1e30)
    v1 = jnp.max(masked, axis=-1, keepdims=True)
    eq1 = (masked == v1).astype(jnp.float32)

    e = jnp.exp(v1 - v0)
    denom = 1.0 + e
    p0 = 1.0 / (denom * jnp.sum(eq0, axis=-1, keepdims=True))
    p1 = e / (denom * jnp.sum(eq1, axis=-1, keepdims=True))
    attn = p0 * eq0 + p1 * eq1
    mem = jnp.dot(attn, mv_ref[...])

    h = _gelu(jnp.dot(z, wh1a_ref[...]) + jnp.dot(mem, wh1b_ref[...])
              + bh1_ref[...])
    head = jnp.dot(h, wh2_ref[...]) + bh2_ref[...]
    out_ref[...] = jax.nn.sigmoid(head)


@jax.jit
def _run(x, W1, b1, W2, b2, memory_keys, memory_values, Wh1, bh1, Wh2, bh2):
    n, d = x.shape
    h0 = W1.shape[1]
    l = W2.shape[1]

    mkT = memory_keys.T
    wh1a = Wh1[:l]
    wh1b = Wh1[l:]

    rep = lambda *shape: pl.BlockSpec(shape, lambda i: (0,) * len(shape))
    out = pl.pallas_call(
        _fused_kernel,
        grid=(n // BN,),
        in_specs=[
            pl.BlockSpec((BN, d), lambda i: (i, 0)),
            rep(d, h0), rep(1, h0), rep(h0, l), rep(1, l),
            rep(l, memory_keys.shape[0]), rep(memory_values.shape[0], l),
            rep(l, h0), rep(l, h0), rep(1, h0), rep(h0, 1), rep(1, 1),
        ],
        out_specs=pl.BlockSpec((BN, 1), lambda i: (i, 0)),
        out_shape=jax.ShapeDtypeStruct((n, 1), jnp.float32),
    )(x, W1, b1.reshape(1, -1), W2, b2.reshape(1, -1), mkT, memory_values,
      wh1a, wh1b, bh1.reshape(1, -1), Wh2, bh2.reshape(1, -1))
    return out.reshape(n)


def kernel(x, W1, b1, W2, b2, memory_keys, memory_values, Wh1, bh1, Wh2, bh2):
    return _run(x, W1, b1, W2, b2, memory_keys, memory_values,
                Wh1, bh1, Wh2, bh2)


# speedup vs baseline: 6.0607x; 1.2875x over previous
import functools

import jax
import jax.numpy as jnp
from jax.experimental import pallas as pl

TAU = 0.7

BN = 1024


_INV_SQRT2 = 0.7071067811865476


def _gelu(v):
    return 0.5 * v * (1.0 + jax.lax.erf(v * _INV_SQRT2))


def _fused_kernel(x_ref, w1_ref, b1_ref, w2_ref, b2_ref, mk_ref, mv_ref,
                  wh1a_ref, wh1b_ref, bh1_ref, wh2_ref, bh2_ref, out_ref):
    x = x_ref[...]
    z1 = _gelu(jnp.dot(x, w1_ref[...]) + b1_ref[...])
    z = _gelu(jnp.dot(z1, w2_ref[...]) + b2_ref[...])

    logits = jnp.dot(z, mk_ref[...]) * (1.0 / TAU)

    v0 = jnp.max(logits, axis=-1, keepdims=True)
    eq0 = (logits == v0).astype(jnp.float32)
    masked = logits - eq0 * jnp.float32(1e30)
    v1 = jnp.max(masked, axis=-1, keepdims=True)
    eq1 = (masked == v1).astype(jnp.float32)

    e = jnp.exp(v1 - v0)
    denom = 1.0 + e
    p0 = 1.0 / (denom * jnp.sum(eq0, axis=-1, keepdims=True))
    p1 = e / (denom * jnp.sum(eq1, axis=-1, keepdims=True))
    attn = p0 * eq0 + p1 * eq1
    mem = jnp.dot(attn, mv_ref[...])

    h = _gelu(jnp.dot(z, wh1a_ref[...]) + jnp.dot(mem, wh1b_ref[...])
              + bh1_ref[...])
    head = jnp.dot(h, wh2_ref[...]) + bh2_ref[...]
    out_ref[...] = jax.nn.sigmoid(head)


@jax.jit
def _run(x, W1, b1, W2, b2, memory_keys, memory_values, Wh1, bh1, Wh2, bh2):
    n, d = x.shape
    h0 = W1.shape[1]
    l = W2.shape[1]

    mkT = memory_keys.T
    wh1a = Wh1[:l]
    wh1b = Wh1[l:]

    rep = lambda *shape: pl.BlockSpec(shape, lambda i: (0,) * len(shape))
    out = pl.pallas_call(
        _fused_kernel,
        grid=(n // BN,),
        in_specs=[
            pl.BlockSpec((BN, d), lambda i: (i, 0)),
            rep(d, h0), rep(1, h0), rep(h0, l), rep(1, l),
            rep(l, memory_keys.shape[0]), rep(memory_values.shape[0], l),
            rep(l, h0), rep(l, h0), rep(1, h0), rep(h0, 1), rep(1, 1),
        ],
        out_specs=pl.BlockSpec((BN, 1), lambda i: (i, 0)),
        out_shape=jax.ShapeDtypeStruct((n, 1), jnp.float32),
    )(x, W1, b1.reshape(1, -1), W2, b2.reshape(1, -1), mkT, memory_values,
      wh1a, wh1b, bh1.reshape(1, -1), Wh2, bh2.reshape(1, -1))
    return out.reshape(n)


def kernel(x, W1, b1, W2, b2, memory_keys, memory_values, Wh1, bh1, Wh2, bh2):
    return _run(x, W1, b1, W2, b2, memory_keys, memory_values,
                Wh1, bh1, Wh2, bh2)


# speedup vs baseline: 6.6273x; 1.0935x over previous
import functools

import jax
import jax.numpy as jnp
from jax.experimental import pallas as pl

TAU = 0.7

BN = 2048


_INV_SQRT2 = 0.7071067811865476


def _gelu(v):
    return 0.5 * v * (1.0 + jax.lax.erf(v * _INV_SQRT2))


def _fused_kernel(x_ref, w1_ref, b1_ref, w2_ref, b2_ref, mk_ref, mv_ref,
                  wh1a_ref, wh1b_ref, bh1_ref, wh2_ref, bh2_ref, out_ref):
    x = x_ref[...]
    z1 = _gelu(jnp.dot(x, w1_ref[...]) + b1_ref[...])
    z = _gelu(jnp.dot(z1, w2_ref[...]) + b2_ref[...])

    logits = jnp.dot(z, mk_ref[...]) * (1.0 / TAU)

    v0 = jnp.max(logits, axis=-1, keepdims=True)
    eq0 = (logits == v0).astype(jnp.float32)
    masked = logits - eq0 * jnp.float32(1e30)
    v1 = jnp.max(masked, axis=-1, keepdims=True)
    eq1 = (masked == v1).astype(jnp.float32)

    e = jnp.exp(v1 - v0)
    denom = 1.0 + e
    p0 = 1.0 / (denom * jnp.sum(eq0, axis=-1, keepdims=True))
    p1 = e / (denom * jnp.sum(eq1, axis=-1, keepdims=True))
    attn = p0 * eq0 + p1 * eq1
    mem = jnp.dot(attn, mv_ref[...])

    h = _gelu(jnp.dot(z, wh1a_ref[...]) + jnp.dot(mem, wh1b_ref[...])
              + bh1_ref[...])
    head = jnp.dot(h, wh2_ref[...]) + bh2_ref[...]
    out_ref[...] = jax.nn.sigmoid(head)


@jax.jit
def _run(x, W1, b1, W2, b2, memory_keys, memory_values, Wh1, bh1, Wh2, bh2):
    n, d = x.shape
    h0 = W1.shape[1]
    l = W2.shape[1]

    mkT = memory_keys.T
    wh1a = Wh1[:l]
    wh1b = Wh1[l:]

    rep = lambda *shape: pl.BlockSpec(shape, lambda i: (0,) * len(shape))
    out = pl.pallas_call(
        _fused_kernel,
        grid=(n // BN,),
        in_specs=[
            pl.BlockSpec((BN, d), lambda i: (i, 0)),
            rep(d, h0), rep(1, h0), rep(h0, l), rep(1, l),
            rep(l, memory_keys.shape[0]), rep(memory_values.shape[0], l),
            rep(l, h0), rep(l, h0), rep(1, h0), rep(h0, 1), rep(1, 1),
        ],
        out_specs=pl.BlockSpec((BN, 1), lambda i: (i, 0)),
        out_shape=jax.ShapeDtypeStruct((n, 1), jnp.float32),
    )(x, W1, b1.reshape(1, -1), W2, b2.reshape(1, -1), mkT, memory_values,
      wh1a, wh1b, bh1.reshape(1, -1), Wh2, bh2.reshape(1, -1))
    return out.reshape(n)


def kernel(x, W1, b1, W2, b2, memory_keys, memory_values, Wh1, bh1, Wh2, bh2):
    return _run(x, W1, b1, W2, b2, memory_keys, memory_values,
                Wh1, bh1, Wh2, bh2)


# speedup vs baseline: 6.7820x; 1.0233x over previous
import functools

import jax
import jax.numpy as jnp
from jax.experimental import pallas as pl

TAU = 0.7

BN = 4096


_INV_SQRT2 = 0.7071067811865476


def _gelu(v):
    return 0.5 * v * (1.0 + jax.lax.erf(v * _INV_SQRT2))


def _fused_kernel(x_ref, w1_ref, b1_ref, w2_ref, b2_ref, mk_ref, mv_ref,
                  wh1a_ref, wh1b_ref, bh1_ref, wh2_ref, bh2_ref, out_ref):
    x = x_ref[...]
    z1 = _gelu(jnp.dot(x, w1_ref[...]) + b1_ref[...])
    z = _gelu(jnp.dot(z1, w2_ref[...]) + b2_ref[...])

    logits = jnp.dot(z, mk_ref[...]) * (1.0 / TAU)

    v0 = jnp.max(logits, axis=-1, keepdims=True)
    eq0 = (logits == v0).astype(jnp.float32)
    masked = logits - eq0 * jnp.float32(1e30)
    v1 = jnp.max(masked, axis=-1, keepdims=True)
    eq1 = (masked == v1).astype(jnp.float32)

    e = jnp.exp(v1 - v0)
    denom = 1.0 + e
    p0 = 1.0 / (denom * jnp.sum(eq0, axis=-1, keepdims=True))
    p1 = e / (denom * jnp.sum(eq1, axis=-1, keepdims=True))
    attn = p0 * eq0 + p1 * eq1
    mem = jnp.dot(attn, mv_ref[...])

    h = _gelu(jnp.dot(z, wh1a_ref[...]) + jnp.dot(mem, wh1b_ref[...])
              + bh1_ref[...])
    head = jnp.dot(h, wh2_ref[...]) + bh2_ref[...]
    out_ref[...] = jax.nn.sigmoid(head)


@jax.jit
def _run(x, W1, b1, W2, b2, memory_keys, memory_values, Wh1, bh1, Wh2, bh2):
    n, d = x.shape
    h0 = W1.shape[1]
    l = W2.shape[1]

    mkT = memory_keys.T
    wh1a = Wh1[:l]
    wh1b = Wh1[l:]

    rep = lambda *shape: pl.BlockSpec(shape, lambda i: (0,) * len(shape))
    out = pl.pallas_call(
        _fused_kernel,
        grid=(n // BN,),
        in_specs=[
            pl.BlockSpec((BN, d), lambda i: (i, 0)),
            rep(d, h0), rep(1, h0), rep(h0, l), rep(1, l),
            rep(l, memory_keys.shape[0]), rep(memory_values.shape[0], l),
            rep(l, h0), rep(l, h0), rep(1, h0), rep(h0, 1), rep(1, 1),
        ],
        out_specs=pl.BlockSpec((BN, 1), lambda i: (i, 0)),
        out_shape=jax.ShapeDtypeStruct((n, 1), jnp.float32),
    )(x, W1, b1.reshape(1, -1), W2, b2.reshape(1, -1), mkT, memory_values,
      wh1a, wh1b, bh1.reshape(1, -1), Wh2, bh2.reshape(1, -1))
    return out.reshape(n)


def kernel(x, W1, b1, W2, b2, memory_keys, memory_values, Wh1, bh1, Wh2, bh2):
    return _run(x, W1, b1, W2, b2, memory_keys, memory_values,
                Wh1, bh1, Wh2, bh2)


# speedup vs baseline: 9.7624x; 1.4395x over previous
import jax
import jax.numpy as jnp
from jax.experimental import pallas as pl

TAU = 0.7

BN = 4096

_INV_SQRT2 = 0.7071067811865476


def _gelu(v):
    return 0.5 * v * (1.0 + jax.lax.erf(v * _INV_SQRT2))


def _fused_kernel(x_ref, w1_ref, b1_ref, w2_ref, b2_ref, mk_ref, mvT_ref,
                  wh1aT_ref, wh1bT_ref, bh1_ref, wh2T_ref, bh2_ref, out_ref):
    x = x_ref[...]
    z1 = _gelu(jnp.dot(x, w1_ref[...]) + b1_ref[...])
    z = _gelu(jnp.dot(z1, w2_ref[...]) + b2_ref[...])

    zT = z.T
    logits = jnp.dot(mk_ref[...], zT) * (1.0 / TAU)

    v0 = jnp.max(logits, axis=0, keepdims=True)
    eq0 = (logits == v0).astype(jnp.float32)
    masked = logits - eq0 * jnp.float32(1e30)
    v1 = jnp.max(masked, axis=0, keepdims=True)
    eq1 = (masked == v1).astype(jnp.float32)

    e = jnp.exp(v1 - v0)
    denom = 1.0 + e
    p0 = 1.0 / (denom * jnp.sum(eq0, axis=0, keepdims=True))
    p1 = e / (denom * jnp.sum(eq1, axis=0, keepdims=True))
    attnT = p0 * eq0 + p1 * eq1
    memT = jnp.dot(mvT_ref[...], attnT)

    hT = _gelu(jnp.dot(wh1aT_ref[...], zT) + jnp.dot(wh1bT_ref[...], memT)
               + bh1_ref[...])
    head = jnp.dot(wh2T_ref[...], hT) + bh2_ref[...]
    out_ref[...] = jax.nn.sigmoid(head)[None]


@jax.jit
def _run(x, W1, b1, W2, b2, memory_keys, memory_values, Wh1, bh1, Wh2, bh2):
    n, d = x.shape
    h0 = W1.shape[1]
    l = W2.shape[1]
    k = memory_keys.shape[0]

    wh1aT = Wh1[:l].T
    wh1bT = Wh1[l:].T

    rep = lambda *shape: pl.BlockSpec(shape, lambda i: (0,) * len(shape))
    out = pl.pallas_call(
        _fused_kernel,
        grid=(n // BN,),
        in_specs=[
            pl.BlockSpec((BN, d), lambda i: (i, 0)),
            rep(d, h0), rep(1, h0), rep(h0, l), rep(1, l),
            rep(k, l), rep(l, k),
            rep(h0, l), rep(h0, l), rep(h0, 1), rep(1, h0), rep(1, 1),
        ],
        out_specs=pl.BlockSpec((1, 1, BN), lambda i: (i, 0, 0)),
        out_shape=jax.ShapeDtypeStruct((n // BN, 1, BN), jnp.float32),
    )(x, W1, b1.reshape(1, -1), W2, b2.reshape(1, -1), memory_keys,
      memory_values.T, wh1aT, wh1bT, bh1.reshape(-1, 1), Wh2.T,
      bh2.reshape(1, 1))
    return out.reshape(n)


def kernel(x, W1, b1, W2, b2, memory_keys, memory_values, Wh1, bh1, Wh2, bh2):
    return _run(x, W1, b1, W2, b2, memory_keys, memory_values,
                Wh1, bh1, Wh2, bh2)
